# baseline (speedup 1.0000x reference)
import functools

import jax
import jax.numpy as jnp
from jax import lax
from jax.experimental import pallas as pl
from jax.experimental.pallas import tpu as pltpu
from jax.experimental.pallas import tpu_sc as plsc

B, H, W = 16, 384, 384
P = 512
IMG = H * W
NS = 16
PPT = (B * P) // NS
CH = 128
NCH = PPT // CH
LANES = 16

_mesh = plsc.VectorSubcoreMesh(
    core_axis_name="c", subcore_axis_name="s", num_cores=1)


@functools.partial(
    pl.kernel,
    out_type=jax.ShapeDtypeStruct((NS, 2, LANES), jnp.float32),
    mesh=_mesh,
    scratch_types=[
        pltpu.VMEM((PPT * 3,), jnp.float32),
        pltpu.VMEM((PPT,), jnp.float32),
        pltpu.VMEM((PPT,), jnp.int32),
        pltpu.VMEM((PPT,), jnp.float32),
        pltpu.VMEM((2, LANES), jnp.float32),
        pltpu.SemaphoreType.DMA,
    ],
)
def _depth_partials(img_hbm, rdepth_hbm, out_hbm,
                    rd_v, depth_v, idx_v, vals_v, part_v, sem):
    sid = lax.axis_index("s")
    pltpu.sync_copy(rdepth_hbm.at[sid], rd_v)

    boff = sid * IMG
    iota = lax.iota(jnp.int32, LANES)
    dnums = lax.GatherDimensionNumbers(
        offset_dims=(), collapsed_slice_dims=(0,), start_index_map=(0,))

    def _pick(v0, v1, v2, off):
        pos = iota * 3 + off
        perm = (pos % LANES).reshape(LANES, 1)
        g0 = lax.gather(v0, perm, dnums, slice_sizes=(1,),
                        mode=lax.GatherScatterMode.PROMISE_IN_BOUNDS)
        g1 = lax.gather(v1, perm, dnums, slice_sizes=(1,),
                        mode=lax.GatherScatterMode.PROMISE_IN_BOUNDS)
        g2 = lax.gather(v2, perm, dnums, slice_sizes=(1,),
                        mode=lax.GatherScatterMode.PROMISE_IN_BOUNDS)
        return jnp.where(pos < LANES, g0,
                         jnp.where(pos < 2 * LANES, g1, g2))

    copies = []
    for j in range(NCH):
        for k8 in range(CH // LANES):
            k = j * (CH // LANES) + k8
            b48 = k * 3 * LANES
            v0 = rd_v[pl.ds(b48, LANES)]
            v1 = rd_v[pl.ds(b48 + LANES, LANES)]
            v2 = rd_v[pl.ds(b48 + 2 * LANES, LANES)]
            r = _pick(v0, v1, v2, 0)
            c = _pick(v0, v1, v2, 1)
            d = _pick(v0, v1, v2, 2)
            idx_v[pl.ds(k * LANES, LANES)] = (
                r.astype(jnp.int32) * W + c.astype(jnp.int32) + boff)
            depth_v[pl.ds(k * LANES, LANES)] = d
        copies.append(pltpu.async_copy(
            img_hbm.at[idx_v.at[pl.ds(j * CH, CH)]],
            vals_v.at[pl.ds(j * CH, CH)], sem))
    for cp in copies:
        cp.wait()

    acc = jnp.zeros((LANES,), jnp.float32)
    cnt = jnp.zeros((LANES,), jnp.float32)
    for k in range(PPT // LANES):
        v = vals_v[pl.ds(k * LANES, LANES)]
        d = depth_v[pl.ds(k * LANES, LANES)]
        m = d > 0.0
        acc = acc + jnp.where(m, jnp.abs(v - d), 0.0)
        cnt = cnt + jnp.where(m, 1.0, 0.0)
    part_v[0, :] = acc
    part_v[1, :] = cnt
    pltpu.sync_copy(part_v, out_hbm.at[sid])


def _finalize_body(p_ref, o_ref):
    p = p_ref[...]
    s = jnp.sum(p[:, 0, :])
    c = jnp.sum(p[:, 1, :])
    loss = jnp.where(c > 0.0, s / jnp.maximum(c, 1.0), 0.0)
    o_ref[...] = jnp.broadcast_to(loss, (1, 1))


_finalize = pl.pallas_call(
    _finalize_body,
    out_shape=jax.ShapeDtypeStruct((1, 1), jnp.float32),
)


def kernel(output, rdepth):
    img = output.reshape(-1)
    partials = _depth_partials(img, rdepth.reshape(B, P * 3))
    s = jnp.sum(partials[:, 0, :])
    c = jnp.sum(partials[:, 1, :])
    return jnp.where(c > 0.0, s / jnp.maximum(c, 1.0), 0.0)


# speedup vs baseline: 1.5653x; 1.5653x over previous
import functools

import jax
import jax.numpy as jnp
from jax import lax
from jax.experimental import pallas as pl
from jax.experimental.pallas import tpu as pltpu
from jax.experimental.pallas import tpu_sc as plsc

B, H, W = 16, 384, 384
P = 512
IMG = H * W
NS = 16
PPT = (B * P) // NS
CH = 128
NCH = PPT // CH
LANES = 16

_mesh = plsc.VectorSubcoreMesh(
    core_axis_name="c", subcore_axis_name="s", num_cores=1)


@functools.partial(
    pl.kernel,
    out_type=jax.ShapeDtypeStruct((NS, 2, LANES), jnp.float32),
    mesh=_mesh,
    scratch_types=[
        pltpu.VMEM((PPT * 3,), jnp.float32),
        pltpu.VMEM((PPT,), jnp.float32),
        pltpu.VMEM((PPT,), jnp.int32),
        pltpu.VMEM((PPT,), jnp.float32),
        pltpu.VMEM((2, LANES), jnp.float32),
        pltpu.SemaphoreType.DMA,
    ],
)
def _depth_partials(img_hbm, rdepth_hbm, out_hbm,
                    rd_v, depth_v, idx_v, vals_v, part_v, sem):
    sid = lax.axis_index("s")
    pltpu.sync_copy(rdepth_hbm.at[sid], rd_v)

    boff = sid * (H // 8)
    iota = lax.iota(jnp.int32, LANES)
    dnums = lax.GatherDimensionNumbers(
        offset_dims=(), collapsed_slice_dims=(0,), start_index_map=(0,))

    def _pick(v0, v1, v2, off):
        pos = iota * 3 + off
        perm = (pos % LANES).reshape(LANES, 1)
        g0 = lax.gather(v0, perm, dnums, slice_sizes=(1,),
                        mode=lax.GatherScatterMode.PROMISE_IN_BOUNDS)
        g1 = lax.gather(v1, perm, dnums, slice_sizes=(1,),
                        mode=lax.GatherScatterMode.PROMISE_IN_BOUNDS)
        g2 = lax.gather(v2, perm, dnums, slice_sizes=(1,),
                        mode=lax.GatherScatterMode.PROMISE_IN_BOUNDS)
        return jnp.where(pos < LANES, g0,
                         jnp.where(pos < 2 * LANES, g1, g2))

    copies = []
    for j in range(NCH):
        for k8 in range(CH // LANES):
            k = j * (CH // LANES) + k8
            b48 = k * 3 * LANES
            v0 = rd_v[pl.ds(b48, LANES)]
            v1 = rd_v[pl.ds(b48 + LANES, LANES)]
            v2 = rd_v[pl.ds(b48 + 2 * LANES, LANES)]
            r = _pick(v0, v1, v2, 0).astype(jnp.int32)
            c = _pick(v0, v1, v2, 1).astype(jnp.int32)
            d = _pick(v0, v1, v2, 2)
            tile = (boff + (r >> 3)) * 3 + (c >> 7)
            idx_v[pl.ds(k * LANES, LANES)] = (
                (tile << 10) + ((r & 7) << 7) + (c & 127))
            depth_v[pl.ds(k * LANES, LANES)] = d
        copies.append(pltpu.async_copy(
            img_hbm.at[idx_v.at[pl.ds(j * CH, CH)]],
            vals_v.at[pl.ds(j * CH, CH)], sem))
    for cp in copies:
        cp.wait()

    acc = jnp.zeros((LANES,), jnp.float32)
    cnt = jnp.zeros((LANES,), jnp.float32)
    for k in range(PPT // LANES):
        v = vals_v[pl.ds(k * LANES, LANES)]
        d = depth_v[pl.ds(k * LANES, LANES)]
        m = d > 0.0
        acc = acc + jnp.where(m, jnp.abs(v - d), 0.0)
        cnt = cnt + jnp.where(m, 1.0, 0.0)
    part_v[0, :] = acc
    part_v[1, :] = cnt
    pltpu.sync_copy(part_v, out_hbm.at[sid])


def _finalize_body(p_ref, o_ref):
    p = p_ref[...]
    s = jnp.sum(p[:, 0, :])
    c = jnp.sum(p[:, 1, :])
    loss = jnp.where(c > 0.0, s / jnp.maximum(c, 1.0), 0.0)
    o_ref[...] = jnp.broadcast_to(loss, (1, 1))


_finalize = pl.pallas_call(
    _finalize_body,
    out_shape=jax.ShapeDtypeStruct((1, 1), jnp.float32),
)


def kernel(output, rdepth):
    img = (output.reshape(B, H // 8, 8, W // 128, 128)
           .transpose(0, 1, 3, 2, 4)
           .reshape(-1))
    partials = _depth_partials(img, rdepth.reshape(B, P * 3))
    return _finalize(partials)[0, 0]


# speedup vs baseline: 1.5777x; 1.0079x over previous
import functools

import jax
import jax.numpy as jnp
from jax import lax
from jax.experimental import pallas as pl
from jax.experimental.pallas import tpu as pltpu
from jax.experimental.pallas import tpu_sc as plsc

B, H, W = 16, 384, 384
P = 512
NS = 16
PPT = (B * P) // NS
CH = 128
NCH = PPT // CH
LANES = 16

_mesh = plsc.VectorSubcoreMesh(
    core_axis_name="c", subcore_axis_name="s", num_cores=1)


@functools.partial(
    pl.kernel,
    out_type=jax.ShapeDtypeStruct((NS, 2, LANES), jnp.float32),
    mesh=_mesh,
    scratch_types=[
        pltpu.VMEM((PPT,), jnp.float32),
        pltpu.VMEM((PPT,), jnp.float32),
        pltpu.VMEM((PPT,), jnp.float32),
        pltpu.VMEM((PPT,), jnp.int32),
        pltpu.VMEM((PPT,), jnp.float32),
        pltpu.VMEM((2, LANES), jnp.float32),
        pltpu.SemaphoreType.DMA,
        pltpu.SemaphoreType.DMA,
    ],
)
def _depth_partials(img_hbm, rcd_hbm, out_hbm,
                    rows_v, cols_v, depth_v, idx_v, vals_v, part_v,
                    in_sem, sem):
    sid = lax.axis_index("s")
    in_cps = [
        pltpu.async_copy(rcd_hbm.at[0, sid], rows_v, in_sem),
        pltpu.async_copy(rcd_hbm.at[1, sid], cols_v, in_sem),
        pltpu.async_copy(rcd_hbm.at[2, sid], depth_v, in_sem),
    ]
    for cp in in_cps:
        cp.wait()

    boff = sid * (H // 8)
    copies = []
    for j in range(NCH):
        for k8 in range(CH // LANES):
            k = j * (CH // LANES) + k8
            r = rows_v[pl.ds(k * LANES, LANES)].astype(jnp.int32)
            c = cols_v[pl.ds(k * LANES, LANES)].astype(jnp.int32)
            tile = (boff + (r >> 3)) * 3 + (c >> 7)
            idx_v[pl.ds(k * LANES, LANES)] = (
                (tile << 10) + ((r & 7) << 7) + (c & 127))
        copies.append(pltpu.async_copy(
            img_hbm.at[idx_v.at[pl.ds(j * CH, CH)]],
            vals_v.at[pl.ds(j * CH, CH)], sem))
    for cp in copies:
        cp.wait()

    acc = jnp.zeros((LANES,), jnp.float32)
    cnt = jnp.zeros((LANES,), jnp.float32)
    for k in range(PPT // LANES):
        v = vals_v[pl.ds(k * LANES, LANES)]
        d = depth_v[pl.ds(k * LANES, LANES)]
        m = d > 0.0
        acc = acc + jnp.where(m, jnp.abs(v - d), 0.0)
        cnt = cnt + jnp.where(m, 1.0, 0.0)
    part_v[0, :] = acc
    part_v[1, :] = cnt
    pltpu.sync_copy(part_v, out_hbm.at[sid])


def _finalize_body(p_ref, o_ref):
    p = p_ref[...]
    s = jnp.sum(p[:, 0, :])
    c = jnp.sum(p[:, 1, :])
    loss = jnp.where(c > 0.0, s / jnp.maximum(c, 1.0), 0.0)
    o_ref[...] = jnp.broadcast_to(loss, (1, 1))


_finalize = pl.pallas_call(
    _finalize_body,
    out_shape=jax.ShapeDtypeStruct((1, 1), jnp.float32),
)


def kernel(output, rdepth):
    img = (output.reshape(B, H // 8, 8, W // 128, 128)
           .transpose(0, 1, 3, 2, 4)
           .reshape(-1))
    rcd = jnp.transpose(rdepth, (2, 0, 1))
    partials = _depth_partials(img, rcd)
    return _finalize(partials)[0, 0]
